# baseline (speedup 1.0000x reference)
import jax
import jax.numpy as jnp
from jax.experimental import pallas as pl

_NUM_NODES = 100000
_HIDDEN = 128
_BLOCK = 10000
_NB0 = 40000 // _BLOCK
_NB1 = 30000 // _BLOCK
_NB2 = 30000 // _BLOCK
_OFF1 = _NB0
_OFF2 = _NB0 + _NB1
_GRID = _NUM_NODES // _BLOCK


def _node_transform_kernel(f0, f1, f2, w0, b0, w1, b1, w2, b2, out):
    s = pl.program_id(0)

    @pl.when(s < _OFF1)
    def _():
        out[...] = jnp.maximum(
            jnp.dot(f0[...].astype(jnp.bfloat16), w0[...].astype(jnp.bfloat16),
                    preferred_element_type=jnp.float32)
            + b0[...], 0.0)

    @pl.when(jnp.logical_and(s >= _OFF1, s < _OFF2))
    def _():
        out[...] = jnp.maximum(
            jnp.dot(f1[...].astype(jnp.bfloat16), w1[...].astype(jnp.bfloat16),
                    preferred_element_type=jnp.float32)
            + b1[...], 0.0)

    @pl.when(s >= _OFF2)
    def _():
        out[...] = jnp.maximum(
            jnp.dot(f2[...].astype(jnp.bfloat16), w2[...].astype(jnp.bfloat16),
                    preferred_element_type=jnp.float32)
            + b2[...], 0.0)


def kernel(feat0, feat1, feat2, W0, b0, W1, b1, W2, b2):
    b0 = b0.reshape(1, _HIDDEN)
    b1 = b1.reshape(1, _HIDDEN)
    b2 = b2.reshape(1, _HIDDEN)
    return pl.pallas_call(
        _node_transform_kernel,
        grid=(_GRID,),
        in_specs=[
            pl.BlockSpec((_BLOCK, 256), lambda s: (jnp.minimum(s, _NB0 - 1), 0)),
            pl.BlockSpec((_BLOCK, 128), lambda s: (jnp.clip(s - _OFF1, 0, _NB1 - 1), 0)),
            pl.BlockSpec((_BLOCK, 64), lambda s: (jnp.clip(s - _OFF2, 0, _NB2 - 1), 0)),
            pl.BlockSpec((256, _HIDDEN), lambda s: (0, 0)),
            pl.BlockSpec((1, _HIDDEN), lambda s: (0, 0)),
            pl.BlockSpec((128, _HIDDEN), lambda s: (0, 0)),
            pl.BlockSpec((1, _HIDDEN), lambda s: (0, 0)),
            pl.BlockSpec((64, _HIDDEN), lambda s: (0, 0)),
            pl.BlockSpec((1, _HIDDEN), lambda s: (0, 0)),
        ],
        out_specs=pl.BlockSpec((_BLOCK, _HIDDEN), lambda s: (s, 0)),
        out_shape=jax.ShapeDtypeStruct((_NUM_NODES, _HIDDEN), jnp.float32),
    )(feat0, feat1, feat2, W0, b0, W1, b1, W2, b2)


# speedup vs baseline: 1.0184x; 1.0184x over previous
import jax
import jax.numpy as jnp
from jax.experimental import pallas as pl

_NUM_NODES = 100000
_HIDDEN = 128
_BLOCK = 10000
_NB0 = 40000 // _BLOCK
_NB1 = 30000 // _BLOCK
_NB2 = 30000 // _BLOCK
_OFF1 = _NB0
_OFF2 = _NB0 + _NB1
_GRID = _NUM_NODES // _BLOCK


def _node_transform_kernel(f0, f1, f2, w0, b0, w1, b1, w2, b2, out):
    s = pl.program_id(0)

    @pl.when(s < _OFF1)
    def _():
        out[...] = jnp.maximum(
            jnp.dot(f0[...], w0[...], preferred_element_type=jnp.float32)
            + b0[...], 0.0)

    @pl.when(jnp.logical_and(s >= _OFF1, s < _OFF2))
    def _():
        out[...] = jnp.maximum(
            jnp.dot(f1[...], w1[...], preferred_element_type=jnp.float32)
            + b1[...], 0.0)

    @pl.when(s >= _OFF2)
    def _():
        out[...] = jnp.maximum(
            jnp.dot(f2[...], w2[...], preferred_element_type=jnp.float32)
            + b2[...], 0.0)


def kernel(feat0, feat1, feat2, W0, b0, W1, b1, W2, b2):
    b0 = b0.reshape(1, _HIDDEN)
    b1 = b1.reshape(1, _HIDDEN)
    b2 = b2.reshape(1, _HIDDEN)
    return pl.pallas_call(
        _node_transform_kernel,
        grid=(_GRID,),
        in_specs=[
            pl.BlockSpec((_BLOCK, 256), lambda s: (jnp.minimum(s, _NB0 - 1), 0)),
            pl.BlockSpec((_BLOCK, 128), lambda s: (jnp.clip(s - _OFF1, 0, _NB1 - 1), 0)),
            pl.BlockSpec((_BLOCK, 64), lambda s: (jnp.clip(s - _OFF2, 0, _NB2 - 1), 0)),
            pl.BlockSpec((256, _HIDDEN), lambda s: (0, 0)),
            pl.BlockSpec((1, _HIDDEN), lambda s: (0, 0)),
            pl.BlockSpec((128, _HIDDEN), lambda s: (0, 0)),
            pl.BlockSpec((1, _HIDDEN), lambda s: (0, 0)),
            pl.BlockSpec((64, _HIDDEN), lambda s: (0, 0)),
            pl.BlockSpec((1, _HIDDEN), lambda s: (0, 0)),
        ],
        out_specs=pl.BlockSpec((_BLOCK, _HIDDEN), lambda s: (s, 0)),
        out_shape=jax.ShapeDtypeStruct((_NUM_NODES, _HIDDEN), jnp.float32),
    )(feat0, feat1, feat2, W0, b0, W1, b1, W2, b2)


# speedup vs baseline: 1.0191x; 1.0007x over previous
import jax
import jax.numpy as jnp
from jax.experimental import pallas as pl
from jax.experimental.pallas import tpu as pltpu

_NUM_NODES = 100000
_HIDDEN = 128
_BLOCK = 10000
_NB0 = 40000 // _BLOCK
_NB1 = 30000 // _BLOCK
_NB2 = 30000 // _BLOCK
_OFF1 = _NB0
_OFF2 = _NB0 + _NB1
_GRID = _NUM_NODES // _BLOCK


def _node_transform_kernel(f0, f1, f2, w0, b0, w1, b1, w2, b2, out):
    s = pl.program_id(0)

    @pl.when(s < _OFF1)
    def _():
        out[...] = jnp.maximum(
            jnp.dot(f0[...], w0[...], preferred_element_type=jnp.float32)
            + b0[...], 0.0)

    @pl.when(jnp.logical_and(s >= _OFF1, s < _OFF2))
    def _():
        out[...] = jnp.maximum(
            jnp.dot(f1[...], w1[...], preferred_element_type=jnp.float32)
            + b1[...], 0.0)

    @pl.when(s >= _OFF2)
    def _():
        out[...] = jnp.maximum(
            jnp.dot(f2[...], w2[...], preferred_element_type=jnp.float32)
            + b2[...], 0.0)


def kernel(feat0, feat1, feat2, W0, b0, W1, b1, W2, b2):
    b0 = b0.reshape(1, _HIDDEN)
    b1 = b1.reshape(1, _HIDDEN)
    b2 = b2.reshape(1, _HIDDEN)
    return pl.pallas_call(
        _node_transform_kernel,
        grid=(_GRID,),
        in_specs=[
            pl.BlockSpec((_BLOCK, 256), lambda s: (jnp.minimum(s, _NB0 - 1), 0)),
            pl.BlockSpec((_BLOCK, 128), lambda s: (jnp.clip(s - _OFF1, 0, _NB1 - 1), 0)),
            pl.BlockSpec((_BLOCK, 64), lambda s: (jnp.clip(s - _OFF2, 0, _NB2 - 1), 0)),
            pl.BlockSpec((256, _HIDDEN), lambda s: (0, 0)),
            pl.BlockSpec((1, _HIDDEN), lambda s: (0, 0)),
            pl.BlockSpec((128, _HIDDEN), lambda s: (0, 0)),
            pl.BlockSpec((1, _HIDDEN), lambda s: (0, 0)),
            pl.BlockSpec((64, _HIDDEN), lambda s: (0, 0)),
            pl.BlockSpec((1, _HIDDEN), lambda s: (0, 0)),
        ],
        out_specs=pl.BlockSpec((_BLOCK, _HIDDEN), lambda s: (s, 0)),
        out_shape=jax.ShapeDtypeStruct((_NUM_NODES, _HIDDEN), jnp.float32),
        compiler_params=pltpu.CompilerParams(
            dimension_semantics=("parallel",)),
    )(feat0, feat1, feat2, W0, b0, W1, b1, W2, b2)
